# baseline (speedup 1.0000x reference)
import functools

import jax
import jax.numpy as jnp
from jax import lax
from jax.experimental import pallas as pl
from jax.experimental.pallas import tpu as pltpu
from jax.experimental.pallas import tpu_sc as plsc

NUM_EMB = 1000000
SUB = 32
BATCH = 16384
NC, NS = 2, 16
NW = NC * NS
BPW = BATCH // NW
HALF = BPW // 2
PACK = 128 // SUB
FULLCOLS = NUM_EMB // 128
TAIL = NUM_EMB - FULLCOLS * 128
TROWS = FULLCOLS * SUB + SUB
SEG = (FULLCOLS + NW - 1) // NW

_mesh = plsc.VectorSubcoreMesh(core_axis_name="c", subcore_axis_name="s")
_params = pltpu.CompilerParams(needs_layout_passes=False)


@functools.partial(
    pl.kernel,
    mesh=_mesh,
    compiler_params=_params,
    out_type=(jax.ShapeDtypeStruct((TROWS, 128), jnp.float32),
              jax.ShapeDtypeStruct((TROWS, 128), jnp.float32)),
    scratch_types=[
        pltpu.VMEM((2, SUB, 128), jnp.float32),
        pltpu.VMEM((2, SUB, 128), jnp.float32),
        pltpu.VMEM((2, SUB, 128), jnp.float32),
        pltpu.VMEM((2, SUB, 128), jnp.float32),
        pltpu.SemaphoreType.DMA,
        pltpu.SemaphoreType.DMA,
        pltpu.SemaphoreType.DMA,
        pltpu.SemaphoreType.DMA,
    ],
)
def _repack(t0_hbm, t1_hbm, e0_hbm, e1_hbm, r0_hbm, r1_hbm,
            vin0, vin1, vout0, vout1, si0, si1, so0, so1):
    wid = lax.axis_index("s") * NC + lax.axis_index("c")
    iota = lax.iota(jnp.int32, 16)
    rowpat = lax.shift_right_logical(iota, 2)
    lanepat = lax.bitwise_and(iota, 3) * SUB
    last = FULLCOLS - 1

    def col_of(cc):
        c = wid * SEG + cc
        return jnp.where(c > last, last, c)

    def permute(vin, vout, slot):
        for m in range(8):
            rowvec = rowpat + 4 * m
            for j in range(SUB):
                vals = vin[slot, j, pl.ds(16 * m, 16)]
                plsc.store_scatter(vout.at[slot], [rowvec, lanepat + j], vals)

    def drain(dst3, sem):
        pltpu.make_async_copy(
            t0_hbm.at[:, pl.ds(0, 128)], dst3.at[0], sem).wait()

    @pl.loop(0, SEG + 1)
    def _cols(cc):
        slot = lax.bitwise_and(cc, 1)
        slotp = 1 - slot

        @pl.when(cc < SEG)
        def _fire_in():
            c = col_of(cc)
            src0 = t0_hbm.at[:, pl.ds(c * 128, 128)]
            src1 = t1_hbm.at[:, pl.ds(c * 128, 128)]

            @pl.when(slot == 0)
            def _():
                pltpu.async_copy(src0, vin0.at[0], si0)
                pltpu.async_copy(src1, vin1.at[0], si0)

            @pl.when(slot == 1)
            def _():
                pltpu.async_copy(src0, vin0.at[1], si1)
                pltpu.async_copy(src1, vin1.at[1], si1)

        @pl.when(cc >= 1)
        def _process():
            cp = col_of(cc - 1)

            @pl.when(slotp == 0)
            def _():
                drain(vin0, si0)
                drain(vin1, si0)

                @pl.when(cc >= 3)
                def _():
                    drain(vout0, so0)
                    drain(vout1, so0)

            @pl.when(slotp == 1)
            def _():
                drain(vin0, si1)
                drain(vin1, si1)

                @pl.when(cc >= 3)
                def _():
                    drain(vout0, so1)
                    drain(vout1, so1)

            permute(vin0, vout0, slotp)
            permute(vin1, vout1, slotp)
            dst0 = r0_hbm.at[pl.ds(cp * SUB, SUB)]
            dst1 = r1_hbm.at[pl.ds(cp * SUB, SUB)]

            @pl.when(slotp == 0)
            def _():
                pltpu.async_copy(vout0.at[0], dst0, so0)
                pltpu.async_copy(vout1.at[0], dst1, so0)

            @pl.when(slotp == 1)
            def _():
                pltpu.async_copy(vout0.at[1], dst0, so1)
                pltpu.async_copy(vout1.at[1], dst1, so1)

    drain(vout0, so0)
    drain(vout1, so0)
    drain(vout0, so1)
    drain(vout1, so1)

    @pl.when(wid == 0)
    def _tail():
        pltpu.sync_copy(e0_hbm, vin0.at[0])
        pltpu.sync_copy(e1_hbm, vin1.at[0])
        permute(vin0, vout0, 0)
        permute(vin1, vout1, 0)
        pltpu.sync_copy(vout0.at[0], r0_hbm.at[pl.ds(FULLCOLS * SUB, SUB)])
        pltpu.sync_copy(vout1.at[0], r1_hbm.at[pl.ds(FULLCOLS * SUB, SUB)])


@functools.partial(
    pl.kernel,
    mesh=_mesh,
    compiler_params=_params,
    out_type=jax.ShapeDtypeStruct((2 * SUB, BATCH), jnp.float32),
    scratch_types=[
        pltpu.VMEM((BPW // 128, 128), jnp.int32),
        pltpu.VMEM((BPW // 128, 128), jnp.int32),
        pltpu.VMEM((HALF, 128), jnp.float32),
        pltpu.VMEM((HALF, 128), jnp.float32),
        pltpu.VMEM((2 * SUB, BPW), jnp.float32),
        pltpu.SemaphoreType.DMA,
    ],
)
def _hash_embed(idx0_hbm, idx1_hbm, t0_hbm, t1_hbm, out_hbm,
                idx0_v, idx1_v, rows0_v, rows1_v, out_v, sem):
    wid = lax.axis_index("s") * NC + lax.axis_index("c")
    pltpu.sync_copy(idx0_hbm.at[wid], idx0_v)
    pltpu.sync_copy(idx1_hbm.at[wid], idx1_v)
    iota = lax.iota(jnp.int32, 16)

    for half in range(2):
        copies = []
        for g in range(HALF // 16):
            j = (half * HALF + g * 16) // 128
            o = (half * HALF + g * 16) % 128
            iv0 = lax.shift_right_logical(idx0_v[j, pl.ds(o, 16)], 2)
            iv1 = lax.shift_right_logical(idx1_v[j, pl.ds(o, 16)], 2)
            copies.append(pltpu.async_copy(
                t0_hbm.at[iv0], rows0_v.at[pl.ds(g * 16, 16)], sem))
            copies.append(pltpu.async_copy(
                t1_hbm.at[iv1], rows1_v.at[pl.ds(g * 16, 16)], sem))
        for c in copies:
            c.wait()
        for g in range(HALF // 16):
            j = (half * HALF + g * 16) // 128
            o = (half * HALF + g * 16) % 128
            rowids = g * 16 + iota
            colvec = half * HALF + g * 16 + iota
            for t, (idx_v, rows_v) in enumerate(
                ((idx0_v, rows0_v), (idx1_v, rows1_v))):
                lbase = lax.bitwise_and(
                    idx_v[j, pl.ds(o, 16)], PACK - 1) * SUB
                for f in range(SUB):
                    vals = plsc.load_gather(rows_v, [rowids, lbase + f])
                    frow = lax.broadcast_in_dim(
                        jnp.int32(t * SUB + f), (16,), ())
                    plsc.store_scatter(out_v, [frow, colvec], vals)

    pltpu.sync_copy(out_v, out_hbm.at[:, pl.ds(wid * BPW, BPW)])


def kernel(indices, table0, table1):
    idx = indices.astype(jnp.int32)
    idx0 = idx[0].reshape(NW, BPW // 128, 128)
    idx1 = idx[1].reshape(NW, BPW // 128, 128)
    e0 = jnp.pad(table0[FULLCOLS * 128:].T, ((0, 0), (0, 128 - TAIL)))
    e1 = jnp.pad(table1[FULLCOLS * 128:].T, ((0, 0), (0, 128 - TAIL)))
    r0, r1 = _repack(table0.T, table1.T, e0, e1)
    out_t = _hash_embed(idx0, idx1, r0, r1)
    return out_t.T


# speedup vs baseline: 1.4272x; 1.4272x over previous
import functools

import jax
import jax.numpy as jnp
from jax import lax
from jax.experimental import pallas as pl
from jax.experimental.pallas import tpu as pltpu
from jax.experimental.pallas import tpu_sc as plsc

NUM_EMB = 1000000
SUB = 32
BATCH = 16384
NC, NS = 2, 16
NW = NC * NS
BPW = BATCH // NW
CHUNK = 128
NCH = BPW // CHUNK
HALF = BPW // 2

_mesh = plsc.VectorSubcoreMesh(core_axis_name="c", subcore_axis_name="s")


@functools.partial(
    pl.kernel,
    mesh=_mesh,
    compiler_params=pltpu.CompilerParams(use_tc_tiling_on_sc=False),
    out_type=jax.ShapeDtypeStruct((BATCH, 2 * SUB), jnp.float32),
    scratch_types=[
        pltpu.VMEM((NCH, CHUNK), jnp.int32),
        pltpu.VMEM((NCH, CHUNK), jnp.int32),
        pltpu.VMEM((HALF, SUB), jnp.float32),
        pltpu.VMEM((HALF, SUB), jnp.float32),
        pltpu.VMEM((HALF, 2 * SUB), jnp.float32),
        pltpu.SemaphoreType.DMA,
    ],
)
def _hash_embed(idx0_hbm, idx1_hbm, t0_hbm, t1_hbm, out_hbm,
                idx0_v, idx1_v, rows0_v, rows1_v, out_v, sem):
    wid = lax.axis_index("s") * NC + lax.axis_index("c")
    base = wid * BPW
    pltpu.sync_copy(idx0_hbm.at[wid], idx0_v)
    pltpu.sync_copy(idx1_hbm.at[wid], idx1_v)
    for half in range(2):
        copies = []
        for jj in range(HALF // 16):
            j = half * (HALF // CHUNK) + jj // (CHUNK // 16)
            k = jj % (CHUNK // 16)
            iv0 = idx0_v[j, pl.ds(k * 16, 16)]
            iv1 = idx1_v[j, pl.ds(k * 16, 16)]
            copies.append(pltpu.async_copy(
                t0_hbm.at[iv0], rows0_v.at[pl.ds(jj * 16, 16)], sem))
            copies.append(pltpu.async_copy(
                t1_hbm.at[iv1], rows1_v.at[pl.ds(jj * 16, 16)], sem))
        for c in copies:
            c.wait()

        @pl.loop(0, HALF)
        def _interleave(r):
            out_v[r, pl.ds(0, 16)] = rows0_v[r, pl.ds(0, 16)]
            out_v[r, pl.ds(16, 16)] = rows0_v[r, pl.ds(16, 16)]
            out_v[r, pl.ds(32, 16)] = rows1_v[r, pl.ds(0, 16)]
            out_v[r, pl.ds(48, 16)] = rows1_v[r, pl.ds(16, 16)]

        pltpu.sync_copy(out_v, out_hbm.at[pl.ds(base + half * HALF, HALF)])


def kernel(indices, table0, table1):
    idx = indices.astype(jnp.int32)
    idx0 = idx[0].reshape(NW, NCH, CHUNK)
    idx1 = idx[1].reshape(NW, NCH, CHUNK)
    return _hash_embed(idx0, idx1, table0, table1)
